# baseline (speedup 1.0000x reference)
import functools

import jax
import jax.numpy as jnp
from jax import lax
from jax.experimental import pallas as pl
from jax.experimental.pallas import tpu as pltpu
from jax.experimental.pallas import tpu_sc as plsc

EMBED_DIM = 64
HALF = 32
NC = 2
NS = 16
NW = NC * NS
BLK = 128
CH = 64


def _make_gather(n_batch: int, n_seq: int, half_vocab: int):
    mesh = plsc.VectorSubcoreMesh(core_axis_name="c", subcore_axis_name="s")

    @functools.partial(
        pl.kernel,
        mesh=mesh,
        out_type=jax.ShapeDtypeStruct((n_batch * n_seq * 2, HALF), jnp.float32),
        scratch_types=[
            pltpu.VMEM((n_seq, BLK), jnp.int32),
            pltpu.VMEM((2 * CH,), jnp.int32),
            pltpu.VMEM((2 * CH, HALF), jnp.float32),
            pltpu.SemaphoreType.DMA,
        ],
        compiler_params=pltpu.CompilerParams(
            use_tc_tiling_on_sc=False, needs_layout_passes=False
        ),
    )
    def gather_kernel(tokens_t_hbm, table_hbm, out_hbm, idx_v, idx32_v, rows_v, sem):
        wid = lax.axis_index("s") * NC + lax.axis_index("c")
        base = wid * BLK
        pltpu.sync_copy(tokens_t_hbm.at[:, pl.ds(base, BLK)], idx_v)
        lanes = lax.iota(jnp.int32, 16)

        def body(c, carry):
            j = c // (BLK // CH)
            h = c % (BLK // CH)
            for g in range(CH // 16):
                t = idx_v[j, pl.ds(h * CH + g * 16, 16)]
                q = (t >= half_vocab).astype(jnp.int32)
                b = (t - half_vocab * q) * 4 + 2 * q
                plsc.store_scatter(idx32_v, [lanes * 2 + g * 32], b)
                plsc.store_scatter(idx32_v, [lanes * 2 + 1 + g * 32], b + 1)
            pltpu.async_copy(table_hbm.at[idx32_v], rows_v, sem).wait()
            pltpu.sync_copy(
                rows_v,
                out_hbm.at[pl.ds(2 * (j * n_batch + base + h * CH), 2 * CH)],
            )
            return carry

        lax.fori_loop(0, n_seq * (BLK // CH), body, 0)

    return gather_kernel


def kernel(tokens, table):
    n_batch, n_seq = tokens.shape
    assert n_batch % NW == 0 and n_batch // NW == BLK
    tokens_t = tokens.T.astype(jnp.int32)
    half = table.shape[0] // 2
    table128 = jax.lax.optimization_barrier(
        table.reshape(2, half, EMBED_DIM).transpose(1, 0, 2).reshape(half, 128)
    )
    table32 = table128.reshape(2 * table.shape[0], HALF)
    out = _make_gather(n_batch, n_seq, half)(tokens_t, table32)
    return out.reshape(n_seq, n_batch, EMBED_DIM).transpose(1, 0, 2)


# speedup vs baseline: 2.2033x; 2.2033x over previous
import functools

import jax
import jax.numpy as jnp
from jax import lax
from jax.experimental import pallas as pl
from jax.experimental.pallas import tpu as pltpu
from jax.experimental.pallas import tpu_sc as plsc

EMBED_DIM = 64
HALF = 32
NC = 2
NS = 16
NW = NC * NS
BLK = 128
CH = 64


def _make_gather(n_batch: int, n_seq: int, half_vocab: int):
    mesh = plsc.VectorSubcoreMesh(core_axis_name="c", subcore_axis_name="s")

    @functools.partial(
        pl.kernel,
        mesh=mesh,
        out_type=jax.ShapeDtypeStruct((n_batch * n_seq * 2, HALF), jnp.float32),
        scratch_types=[
            pltpu.VMEM((n_seq, BLK), jnp.int32),
            pltpu.VMEM((2 * CH,), jnp.int32),
            pltpu.VMEM((2 * CH, HALF), jnp.float32),
            pltpu.SemaphoreType.DMA,
        ],
        compiler_params=pltpu.CompilerParams(
            use_tc_tiling_on_sc=False, needs_layout_passes=False
        ),
    )
    def gather_kernel(tokens_t_hbm, table_hbm, out_hbm, idx_v, idx32_v, rows_v, sem):
        wid = lax.axis_index("s") * NC + lax.axis_index("c")
        base = wid * BLK
        pltpu.sync_copy(tokens_t_hbm.at[:, pl.ds(base, BLK)], idx_v)
        lanes = lax.iota(jnp.int32, 16)

        def body(c, carry):
            j = c // (BLK // CH)
            h = c % (BLK // CH)
            for g in range(CH // 16):
                t = idx_v[j, pl.ds(h * CH + g * 16, 16)]
                q = (t >= half_vocab).astype(jnp.int32)
                b = (t - half_vocab * q) * 4 + 2 * q
                plsc.store_scatter(idx32_v, [lanes * 2 + g * 32], b)
                plsc.store_scatter(idx32_v, [lanes * 2 + 1 + g * 32], b + 1)
            pltpu.async_copy(table_hbm.at[idx32_v], rows_v, sem).wait()
            pltpu.sync_copy(
                rows_v,
                out_hbm.at[pl.ds(2 * (j * n_batch + base + h * CH), 2 * CH)],
            )
            return carry

        lax.fori_loop(0, n_seq * (BLK // CH), body, 0)

    return gather_kernel


def kernel(tokens, table):
    n_batch, n_seq = tokens.shape
    assert n_batch % NW == 0 and n_batch // NW == BLK
    tokens_t = tokens.T.astype(jnp.int32)
    half = table.shape[0] // 2
    table128 = jax.lax.optimization_barrier(
        jnp.pad(table[:half], ((0, 0), (0, EMBED_DIM)))
        + jnp.pad(table[half:], ((0, 0), (EMBED_DIM, 0)))
    )
    table32 = table128.reshape(2 * table.shape[0], HALF)
    out = _make_gather(n_batch, n_seq, half)(tokens_t, table32)
    return out.reshape(n_seq, n_batch, EMBED_DIM).transpose(1, 0, 2)
